# baseline (speedup 1.0000x reference)
import functools

import jax
import jax.numpy as jnp
from jax import lax
from jax.experimental import pallas as pl
from jax.experimental.pallas import tpu as pltpu
from jax.experimental.pallas import tpu_sc as plsc

N = 10000
E = 320000
IN = 128
OUT = 128
K = 4
DIM = 3
VOX = 2
G = 16
NCL = G ** 3
KK = K ** DIM

NC = 2
NS = 16
NW = NC * NS
EPT = E // NW
CH = 16
SUP = 400
SUPP = 512
NFLD = 10
CPB = SUP // CH
BLKS = EPT // SUP
NBLK = E // SUP
EBW = NFLD * SUPP
NPAD = 10240
NPT = NPAD // NS
EPAD = 655360
EC2 = EPAD // NW
EB1 = 1024
EOB = EC2 // EB1

_CORNER_OFF = [(s & 1) + ((s >> 1) & 1) * K + ((s >> 2) & 1) * K * K
               for s in range(8)]


def _xtrans_body(x_ref, w_ref, o_ref):
    o_ref[...] = jnp.dot(x_ref[...], w_ref[...],
                         preferred_element_type=jnp.float32)


def _xtrans(x, Wf):
    BN = 1000
    BC = 2048
    return pl.pallas_call(
        _xtrans_body,
        grid=(N // BN, (KK * OUT) // BC),
        in_specs=[
            pl.BlockSpec((BN, IN), lambda n, c: (n, 0)),
            pl.BlockSpec((IN, BC), lambda n, c: (0, c)),
        ],
        out_specs=pl.BlockSpec((BN, BC), lambda n, c: (n, c)),
        out_shape=jax.ShapeDtypeStruct((N, KK * OUT), jnp.float32),
    )(x, Wf)


def _edgeprep_body(att_ref, src_ref, dst_ref, out_ref):
    f = []
    b = []
    for d in range(DIM):
        u = jnp.clip(att_ref[d, :], 0.0, 1.0) * (K - 1)
        u = jnp.minimum(u, K - 1 - 1e-6)
        bot = jnp.floor(u)
        f.append(u - bot)
        b.append(bot.astype(jnp.int32))
    base = b[0] + b[1] * K + b[2] * (K * K)
    gbase = lax.bitcast_convert_type(src_ref[0, :] * KK + base, jnp.float32)
    dstf = lax.bitcast_convert_type(dst_ref[0, :], jnp.float32)
    nsub = out_ref.shape[0]
    for k in range(nsub):
        out_ref[k, 0, :SUP] = gbase[k * SUP:(k + 1) * SUP]
        out_ref[k, 1, :SUP] = dstf[k * SUP:(k + 1) * SUP]
        out_ref[k, :, SUP:] = jnp.zeros((NFLD, SUPP - SUP), jnp.float32)
    for s in range(8):
        w = jnp.ones_like(f[0])
        for d in range(DIM):
            bit = (s >> d) & 1
            w = w * (f[d] if bit else (1.0 - f[d]))
        for k in range(nsub):
            out_ref[k, 2 + s, :SUP] = w[k * SUP:(k + 1) * SUP]


def _edgeprep(attT, srcv, dstv):
    BE = 16000
    return pl.pallas_call(
        _edgeprep_body,
        grid=(E // BE,),
        in_specs=[
            pl.BlockSpec((DIM, BE), lambda e: (0, e)),
            pl.BlockSpec((1, BE), lambda e: (0, e)),
            pl.BlockSpec((1, BE), lambda e: (0, e)),
        ],
        out_specs=pl.BlockSpec((BE // SUP, NFLD, SUPP), lambda e: (e, 0, 0)),
        out_shape=jax.ShapeDtypeStruct((NBLK, NFLD, SUPP), jnp.float32),
    )(attT, srcv, dstv)


def _edge_sc_body(edata_hbm, xt_hbm, zeros2_hbm, zeros1_hbm, cid_hbm, ei_hbm,
                  out_hbm, outdeg_hbm, outrm_hbm,
                  ebuf, idx_v, dst_v, rows_v, msg_v, ones_v, rm_v,
                  agg_sh, deg_sh, esem, gsem):
    cidx = lax.axis_index("c")
    sidx = lax.axis_index("s")
    wid = sidx * NC + cidx

    pltpu.sync_copy(zeros2_hbm.at[pl.ds(sidx * NPT, NPT)],
                    agg_sh.at[pl.ds(sidx * NPT, NPT)])
    pltpu.sync_copy(zeros1_hbm.at[pl.ds(sidx * NPT, NPT)],
                    deg_sh.at[pl.ds(sidx * NPT, NPT)])
    ones_v[...] = jnp.full((CH,), 1.0, jnp.float32)
    plsc.subcore_barrier()

    def fld(eoff, f, t):
        return ebuf[pl.ds(eoff + f * SUPP + t * CH, CH)]

    def build_idx(eoff, t, half):
        gb = lax.bitcast_convert_type(fld(eoff, 0, t), jnp.int32)
        off = pl.multiple_of(half * 128, 128)
        for s in range(8):
            idx_v[pl.ds(off + s * CH, CH)] = gb + _CORNER_OFF[s]

    def issue_gather(half):
        off = pl.multiple_of(half * 128, 128)
        pltpu.async_copy(xt_hbm.at[idx_v.at[pl.ds(off, 128)]],
                         rows_v.at[pl.ds(off, 128)], gsem)

    def wait_gather(half):
        off = pl.multiple_of(half * 128, 128)
        pltpu.make_async_copy(xt_hbm.at[idx_v.at[pl.ds(off, 128)]],
                              rows_v.at[pl.ds(off, 128)], gsem).wait()

    def eslot(b):
        return pl.multiple_of((b % 2) * EBW, 128)

    pltpu.async_copy(edata_hbm.at[pl.ds(wid * BLKS * EBW, EBW)],
                     ebuf.at[pl.ds(0, EBW)], esem)

    def block_body(b, _):
        eoff = eslot(b)
        pltpu.make_async_copy(
            edata_hbm.at[pl.ds((wid * BLKS + b) * EBW, EBW)],
            ebuf.at[pl.ds(eoff, EBW)], esem).wait()

        @pl.when(b + 1 < BLKS)
        def _():
            pltpu.async_copy(
                edata_hbm.at[pl.ds((wid * BLKS + b + 1) * EBW, EBW)],
                ebuf.at[pl.ds(eslot(b + 1), EBW)], esem)

        build_idx(eoff, 0, 0)
        issue_gather(0)

        def chunk_body(t, _):
            half = t % 2
            roff = pl.multiple_of(half * 128, 128)
            wait_gather(half)

            @pl.when(t + 1 < CPB)
            def _():
                build_idx(eoff, t + 1, 1 - half)
                issue_gather(1 - half)

            dst_v[...] = lax.bitcast_convert_type(fld(eoff, 1, t), jnp.int32)
            wvs = [fld(eoff, 2 + s, t) for s in range(8)]
            for i in range(CH):
                m = [jnp.zeros((16,), jnp.float32) for _ in range(8)]
                for s in range(8):
                    w = wvs[s][i]
                    r0 = roff + s * CH + i
                    for j in range(8):
                        m[j] = m[j] + w * rows_v[r0, pl.ds(j * 16, 16)]
                for j in range(8):
                    msg_v[i, pl.ds(j * 16, 16)] = m[j]
            pltpu.sync_copy(msg_v, agg_sh.at[dst_v], add=True)
            pltpu.sync_copy(ones_v, deg_sh.at[dst_v], add=True)
            return 0

        lax.fori_loop(0, CPB, chunk_body, 0)
        return 0

    lax.fori_loop(0, BLKS, block_body, 0)

    plsc.subcore_barrier()
    pltpu.sync_copy(agg_sh.at[pl.ds(sidx * NPT, NPT)],
                    out_hbm.at[cidx, pl.ds(sidx * NPT, NPT)])
    pltpu.sync_copy(deg_sh.at[pl.ds(sidx * NPT, NPT)],
                    outdeg_hbm.at[pl.ds(cidx * NPAD + sidx * NPT, NPT)])

    rmbase = wid * EC2

    def rm_batch(b, _):
        o = rmbase + b * EB1
        pltpu.sync_copy(ei_hbm.at[pl.ds(o, EB1)], rm_v.at[pl.ds(0, EB1)])
        copies = [
            pltpu.async_copy(cid_hbm.at[rm_v.at[pl.ds(k * 128, 128)]],
                             rm_v.at[pl.ds(EB1 + k * 128, 128)], gsem)
            for k in range(EB1 // 128)
        ]
        for cp in copies:
            cp.wait()
        pltpu.sync_copy(rm_v.at[pl.ds(EB1, EB1)], outrm_hbm.at[pl.ds(o, EB1)])
        return 0

    lax.fori_loop(0, EOB, rm_batch, 0)


def _edge_sc(edata_flat, xt_flat, zeros2, zeros1, cid, ei_flat_padded):
    mesh = plsc.VectorSubcoreMesh(core_axis_name="c", subcore_axis_name="s")
    f = functools.partial(
        pl.kernel,
        mesh=mesh,
        out_type=[
            jax.ShapeDtypeStruct((NC, NPAD, OUT), jnp.float32),
            jax.ShapeDtypeStruct((NC * NPAD,), jnp.float32),
            jax.ShapeDtypeStruct((EPAD,), jnp.int32),
        ],
        scratch_types=[
            pltpu.VMEM((2 * EBW,), jnp.float32),
            pltpu.VMEM((2 * 8 * CH,), jnp.int32),
            pltpu.VMEM((CH,), jnp.int32),
            pltpu.VMEM((2 * 8 * CH, OUT), jnp.float32),
            pltpu.VMEM((CH, OUT), jnp.float32),
            pltpu.VMEM((CH,), jnp.float32),
            pltpu.VMEM((2 * EB1,), jnp.int32),
            pltpu.VMEM_SHARED((NPAD, OUT), jnp.float32),
            pltpu.VMEM_SHARED((NPAD,), jnp.float32),
            pltpu.SemaphoreType.DMA,
            pltpu.SemaphoreType.DMA,
        ],
    )(_edge_sc_body)
    return f(edata_flat, xt_flat, zeros2, zeros1, cid, ei_flat_padded)


def _post_body(p_ref, rdeg_ref, x_ref, wr_ref, bias_ref, gamma_ref, beta_ref,
               pos_ref, cid_ref, xp_ref, pp_ref,
               xn_ref, xpa_ref, ppa_ref):
    msg = p_ref[0, :N, :] + p_ref[1, :N, :]
    agg = msg * rdeg_ref[...]
    out = agg + jnp.dot(x_ref[...], wr_ref[...],
                        preferred_element_type=jnp.float32) + bias_ref[0]
    out = jnp.where(out > 0, out, jnp.exp(jnp.minimum(out, 0.0)) - 1.0)
    mean = jnp.mean(out, axis=0)
    var = jnp.mean(out * out, axis=0) - mean * mean
    rstd = lax.rsqrt(var + 1e-5)
    xn_ref[...] = (out - mean) * rstd * gamma_ref[0] + beta_ref[0]

    xpa_ref[...] = jnp.full((NCL, OUT), -jnp.inf, dtype=jnp.float32)
    ppa_ref[...] = jnp.zeros((NCL, 8), dtype=jnp.float32)

    def body(i, _):
        c = cid_ref[i]
        row = xn_ref[pl.ds(i, 1), :]
        xpa_ref[pl.ds(c, 1), :] = jnp.maximum(xpa_ref[pl.ds(c, 1), :], row)
        ppa_ref[pl.ds(c, 1), :] += pos_ref[pl.ds(i, 1), :]
        return 0

    lax.fori_loop(0, N, body, 0)
    cnt = ppa_ref[:, 3][:, None]
    xp_ref[...] = jnp.where(cnt > 0, xpa_ref[...], 0.0)
    pp_ref[...] = ppa_ref[...] / jnp.maximum(cnt, 1.0)


def _post(partials, rdeg, x, W_root, bias, gamma, beta, pos8, cid):
    return pl.pallas_call(
        _post_body,
        in_specs=[
            pl.BlockSpec(memory_space=pltpu.VMEM),
            pl.BlockSpec(memory_space=pltpu.VMEM),
            pl.BlockSpec(memory_space=pltpu.VMEM),
            pl.BlockSpec(memory_space=pltpu.VMEM),
            pl.BlockSpec(memory_space=pltpu.VMEM),
            pl.BlockSpec(memory_space=pltpu.VMEM),
            pl.BlockSpec(memory_space=pltpu.VMEM),
            pl.BlockSpec(memory_space=pltpu.VMEM),
            pl.BlockSpec(memory_space=pltpu.SMEM),
        ],
        out_specs=[
            pl.BlockSpec(memory_space=pltpu.VMEM),
            pl.BlockSpec(memory_space=pltpu.VMEM),
        ],
        out_shape=[
            jax.ShapeDtypeStruct((NCL, OUT), jnp.float32),
            jax.ShapeDtypeStruct((NCL, 8), jnp.float32),
        ],
        scratch_shapes=[
            pltpu.VMEM((N, OUT), jnp.float32),
            pltpu.VMEM((NCL, OUT), jnp.float32),
            pltpu.VMEM((NCL, 8), jnp.float32),
        ],
    )(partials, rdeg, x, W_root, bias[None, :], gamma[None, :], beta[None, :],
      pos8, cid)


def kernel(x, edge_index, edge_attr, pos, batch, W, W_root, bias, gamma, beta):
    src = edge_index[0]
    dst = edge_index[1]

    Wf = jnp.transpose(W, (1, 0, 2)).reshape(IN, KK * OUT)
    x_trans = _xtrans(x, Wf)
    xt_flat = x_trans.reshape(N * KK, OUT)
    edata = _edgeprep(edge_attr.T, src[None, :], dst[None, :])

    c = jnp.clip(jnp.floor(pos / VOX).astype(jnp.int32), 0, G - 1)
    cid = c[:, 0] + c[:, 1] * G + c[:, 2] * (G * G)
    ei_flat = jnp.pad(edge_index.reshape(2 * E), (0, EPAD - 2 * E))

    zeros2 = jnp.zeros((NPAD, OUT), jnp.float32)
    zeros1 = jnp.zeros((NPAD,), jnp.float32)
    partials, degp, rm = _edge_sc(edata.reshape(NBLK * EBW), xt_flat,
                                  zeros2, zeros1, cid, ei_flat)
    new_ei = rm[:2 * E].reshape(2, E)
    degp = degp.reshape(NC, NPAD)
    rdeg = (1.0 / jnp.maximum(degp[0, :N] + degp[1, :N], 1.0))[:, None]

    pos8 = jnp.concatenate(
        [pos, jnp.ones((N, 1), jnp.float32), jnp.zeros((N, 4), jnp.float32)],
        axis=1)

    xp, pp8 = _post(partials, rdeg, x, W_root, bias, gamma, beta, pos8, cid)
    pp = pp8[:, :DIM]
    return xp, pp, new_ei


# speedup vs baseline: 1.0470x; 1.0470x over previous
import functools

import jax
import jax.numpy as jnp
from jax import lax
from jax.experimental import pallas as pl
from jax.experimental.pallas import tpu as pltpu
from jax.experimental.pallas import tpu_sc as plsc

N = 10000
E = 320000
IN = 128
OUT = 128
K = 4
DIM = 3
VOX = 2
G = 16
NCL = G ** 3
KK = K ** DIM

NC = 2
NS = 16
NW = NC * NS
EPT = E // NW
CH = 16
SUP = 400
SUPP = 512
NFLD = 10
CPB = SUP // CH
BLKS = EPT // SUP
NBLK = E // SUP
EBW = NFLD * SUPP
NPAD = 10240
NPT = NPAD // NS
EPAD = 655360
EC2 = EPAD // NW
EB1 = 1024
EOB = EC2 // EB1

_CORNER_OFF = [(s & 1) + ((s >> 1) & 1) * K + ((s >> 2) & 1) * K * K
               for s in range(8)]


def _xtrans_body(x_ref, w_ref, o_ref):
    o_ref[...] = jnp.dot(x_ref[...], w_ref[...],
                         preferred_element_type=jnp.float32)


def _xtrans(x, Wf):
    BN = 1000
    BC = 2048
    return pl.pallas_call(
        _xtrans_body,
        grid=(N // BN, (KK * OUT) // BC),
        in_specs=[
            pl.BlockSpec((BN, IN), lambda n, c: (n, 0)),
            pl.BlockSpec((IN, BC), lambda n, c: (0, c)),
        ],
        out_specs=pl.BlockSpec((BN, BC), lambda n, c: (n, c)),
        out_shape=jax.ShapeDtypeStruct((N, KK * OUT), jnp.float32),
    )(x, Wf)


def _edgeprep_body(att_ref, src_ref, dst_ref, out_ref):
    f = []
    b = []
    for d in range(DIM):
        u = jnp.clip(att_ref[d, :], 0.0, 1.0) * (K - 1)
        u = jnp.minimum(u, K - 1 - 1e-6)
        bot = jnp.floor(u)
        f.append(u - bot)
        b.append(bot.astype(jnp.int32))
    base = b[0] + b[1] * K + b[2] * (K * K)
    gbase = lax.bitcast_convert_type(src_ref[0, :] * KK + base, jnp.float32)
    dstf = lax.bitcast_convert_type(dst_ref[0, :], jnp.float32)
    nsub = out_ref.shape[0]
    for k in range(nsub):
        out_ref[k, 0, :SUP] = gbase[k * SUP:(k + 1) * SUP]
        out_ref[k, 1, :SUP] = dstf[k * SUP:(k + 1) * SUP]
        out_ref[k, :, SUP:] = jnp.zeros((NFLD, SUPP - SUP), jnp.float32)
    for s in range(8):
        w = jnp.ones_like(f[0])
        for d in range(DIM):
            bit = (s >> d) & 1
            w = w * (f[d] if bit else (1.0 - f[d]))
        for k in range(nsub):
            out_ref[k, 2 + s, :SUP] = w[k * SUP:(k + 1) * SUP]


def _edgeprep(attT, srcv, dstv):
    BE = 16000
    return pl.pallas_call(
        _edgeprep_body,
        grid=(E // BE,),
        in_specs=[
            pl.BlockSpec((DIM, BE), lambda e: (0, e)),
            pl.BlockSpec((1, BE), lambda e: (0, e)),
            pl.BlockSpec((1, BE), lambda e: (0, e)),
        ],
        out_specs=pl.BlockSpec((BE // SUP, NFLD, SUPP), lambda e: (e, 0, 0)),
        out_shape=jax.ShapeDtypeStruct((NBLK, NFLD, SUPP), jnp.float32),
    )(attT, srcv, dstv)


def _edge_sc_body(edata_hbm, xt_hbm, zeros2_hbm, zeros1_hbm,
                  out_hbm, outdeg_hbm,
                  ebuf, idx_v, dst_v, rows_v, msg_v, ones_v,
                  agg_sh, deg_sh, esem, gsem, ssem):
    cidx = lax.axis_index("c")
    sidx = lax.axis_index("s")
    wid = sidx * NC + cidx

    pltpu.sync_copy(zeros2_hbm.at[pl.ds(sidx * NPT, NPT)],
                    agg_sh.at[pl.ds(sidx * NPT, NPT)])
    pltpu.sync_copy(zeros1_hbm.at[pl.ds(sidx * NPT, NPT)],
                    deg_sh.at[pl.ds(sidx * NPT, NPT)])
    ones_v[...] = jnp.full((CH,), 1.0, jnp.float32)
    plsc.subcore_barrier()

    def fld(eoff, f, t):
        return ebuf[pl.ds(eoff + f * SUPP + t * CH, CH)]

    def build_idx(eoff, t, half):
        gb = lax.bitcast_convert_type(fld(eoff, 0, t), jnp.int32)
        off = pl.multiple_of(half * 128, 128)
        for s in range(8):
            idx_v[pl.ds(off + s * CH, CH)] = gb + _CORNER_OFF[s]

    def issue_gather(half):
        off = pl.multiple_of(half * 128, 128)
        pltpu.async_copy(xt_hbm.at[idx_v.at[pl.ds(off, 128)]],
                         rows_v.at[pl.ds(off, 128)], gsem)

    def wait_gather(half):
        off = pl.multiple_of(half * 128, 128)
        pltpu.make_async_copy(xt_hbm.at[idx_v.at[pl.ds(off, 128)]],
                              rows_v.at[pl.ds(off, 128)], gsem).wait()

    def wait_scatter(h):
        hoff = pl.multiple_of(h * CH, 8)
        pltpu.make_async_copy(msg_v.at[pl.ds(hoff, CH)],
                              agg_sh.at[dst_v.at[h]], ssem).wait()
        pltpu.make_async_copy(ones_v,
                              deg_sh.at[dst_v.at[h]], ssem).wait()

    def eslot(b):
        return pl.multiple_of((b % 2) * EBW, 128)

    pltpu.async_copy(edata_hbm.at[pl.ds(wid * BLKS * EBW, EBW)],
                     ebuf.at[pl.ds(0, EBW)], esem)

    def block_body(b, _):
        eoff = eslot(b)
        pltpu.make_async_copy(
            edata_hbm.at[pl.ds((wid * BLKS + b) * EBW, EBW)],
            ebuf.at[pl.ds(eoff, EBW)], esem).wait()

        @pl.when(b + 1 < BLKS)
        def _():
            pltpu.async_copy(
                edata_hbm.at[pl.ds((wid * BLKS + b + 1) * EBW, EBW)],
                ebuf.at[pl.ds(eslot(b + 1), EBW)], esem)

        build_idx(eoff, 0, 0)
        issue_gather(0)

        def chunk_body(t, _):
            half = t % 2
            roff = pl.multiple_of(half * 128, 128)
            wait_gather(half)

            @pl.when(t + 1 < CPB)
            def _():
                build_idx(eoff, t + 1, 1 - half)
                issue_gather(1 - half)

            roff8 = pl.multiple_of(half * CH, 8)
            dst_v[half, :] = lax.bitcast_convert_type(
                fld(eoff, 1, t), jnp.int32)
            wvs = [fld(eoff, 2 + s, t) for s in range(8)]
            for i in range(CH):
                m = [jnp.zeros((16,), jnp.float32) for _ in range(8)]
                for s in range(8):
                    w = wvs[s][i]
                    r0 = roff + s * CH + i
                    for j in range(8):
                        m[j] = m[j] + w * rows_v[r0, pl.ds(j * 16, 16)]
                for j in range(8):
                    msg_v[roff8 + i, pl.ds(j * 16, 16)] = m[j]

            @pl.when(t > 0)
            def _():
                wait_scatter(1 - half)

            pltpu.async_copy(msg_v.at[pl.ds(roff8, CH)],
                             agg_sh.at[dst_v.at[half]], ssem, add=True)
            pltpu.async_copy(ones_v,
                             deg_sh.at[dst_v.at[half]], ssem, add=True)
            return 0

        lax.fori_loop(0, CPB, chunk_body, 0)
        wait_scatter((CPB - 1) % 2)
        return 0

    lax.fori_loop(0, BLKS, block_body, 0)

    plsc.subcore_barrier()
    pltpu.sync_copy(agg_sh.at[pl.ds(sidx * NPT, NPT)],
                    out_hbm.at[cidx, pl.ds(sidx * NPT, NPT)])
    pltpu.sync_copy(deg_sh.at[pl.ds(sidx * NPT, NPT)],
                    outdeg_hbm.at[pl.ds(cidx * NPAD + sidx * NPT, NPT)])


def _edge_sc(edata_flat, xt_flat, zeros2, zeros1):
    mesh = plsc.VectorSubcoreMesh(core_axis_name="c", subcore_axis_name="s")
    f = functools.partial(
        pl.kernel,
        mesh=mesh,
        out_type=[
            jax.ShapeDtypeStruct((NC, NPAD, OUT), jnp.float32),
            jax.ShapeDtypeStruct((NC * NPAD,), jnp.float32),
        ],
        scratch_types=[
            pltpu.VMEM((2 * EBW,), jnp.float32),
            pltpu.VMEM((2 * 8 * CH,), jnp.int32),
            pltpu.VMEM((2, CH), jnp.int32),
            pltpu.VMEM((2 * 8 * CH, OUT), jnp.float32),
            pltpu.VMEM((2 * CH, OUT), jnp.float32),
            pltpu.VMEM((CH,), jnp.float32),
            pltpu.VMEM_SHARED((NPAD, OUT), jnp.float32),
            pltpu.VMEM_SHARED((NPAD,), jnp.float32),
            pltpu.SemaphoreType.DMA,
            pltpu.SemaphoreType.DMA,
            pltpu.SemaphoreType.DMA,
        ],
    )(_edge_sc_body)
    return f(edata_flat, xt_flat, zeros2, zeros1)


def _remap_body(cid_hbm, ei_hbm, out_hbm, idx_v, out_v, sem):
    cidx = lax.axis_index("c")
    sidx = lax.axis_index("s")
    wid = sidx * NC + cidx
    base = wid * EC2

    def batch(b, _):
        o = base + b * EB1
        pltpu.sync_copy(ei_hbm.at[pl.ds(o, EB1)], idx_v)
        copies = [
            pltpu.async_copy(cid_hbm.at[idx_v.at[pl.ds(k * 128, 128)]],
                             out_v.at[pl.ds(k * 128, 128)], sem)
            for k in range(EB1 // 128)
        ]
        for cp in copies:
            cp.wait()
        pltpu.sync_copy(out_v, out_hbm.at[pl.ds(o, EB1)])
        return 0

    lax.fori_loop(0, EOB, batch, 0)


def _remap(cid, ei_flat_padded):
    mesh = plsc.VectorSubcoreMesh(core_axis_name="c", subcore_axis_name="s")
    f = functools.partial(
        pl.kernel,
        mesh=mesh,
        out_type=jax.ShapeDtypeStruct((EPAD,), jnp.int32),
        scratch_types=[
            pltpu.VMEM((EB1,), jnp.int32),
            pltpu.VMEM((EB1,), jnp.int32),
            pltpu.SemaphoreType.DMA,
        ],
    )(_remap_body)
    return f(cid, ei_flat_padded)


def _post_body(p_ref, rdeg_ref, x_ref, wr_ref, bias_ref, gamma_ref, beta_ref,
               pos_ref, cid_ref, xp_ref, pp_ref,
               xn_ref, xpa_ref, ppa_ref):
    msg = p_ref[0, :N, :] + p_ref[1, :N, :]
    agg = msg * rdeg_ref[...]
    out = agg + jnp.dot(x_ref[...], wr_ref[...],
                        preferred_element_type=jnp.float32) + bias_ref[0]
    out = jnp.where(out > 0, out, jnp.exp(jnp.minimum(out, 0.0)) - 1.0)
    mean = jnp.mean(out, axis=0)
    var = jnp.mean(out * out, axis=0) - mean * mean
    rstd = lax.rsqrt(var + 1e-5)
    xn_ref[...] = (out - mean) * rstd * gamma_ref[0] + beta_ref[0]

    xpa_ref[...] = jnp.full((NCL, 1, OUT), -jnp.inf, dtype=jnp.float32)
    ppa_ref[...] = jnp.zeros((NCL, 8), dtype=jnp.float32)

    def body(a, _):
        a8 = pl.multiple_of(8 * a, 8)
        rows8 = xn_ref[pl.ds(a8, 8), :]
        pos88 = pos_ref[pl.ds(a8, 8), :]
        for bs in range(8):
            c = cid_ref[8 * a + bs]
            row = rows8[bs:bs + 1, :]
            xpa_ref[c, :, :] = jnp.maximum(xpa_ref[c, :, :], row)
            ppa_ref[pl.ds(c, 1), :] += pos88[bs:bs + 1, :]
        return 0

    lax.fori_loop(0, N // 8, body, 0)
    cnt = ppa_ref[:, 3][:, None]
    xp_ref[...] = jnp.where(cnt > 0, xpa_ref[:, 0, :], 0.0)
    pp_ref[...] = ppa_ref[...] / jnp.maximum(cnt, 1.0)


def _post(partials, rdeg, x, W_root, bias, gamma, beta, pos8, cid):
    return pl.pallas_call(
        _post_body,
        in_specs=[
            pl.BlockSpec(memory_space=pltpu.VMEM),
            pl.BlockSpec(memory_space=pltpu.VMEM),
            pl.BlockSpec(memory_space=pltpu.VMEM),
            pl.BlockSpec(memory_space=pltpu.VMEM),
            pl.BlockSpec(memory_space=pltpu.VMEM),
            pl.BlockSpec(memory_space=pltpu.VMEM),
            pl.BlockSpec(memory_space=pltpu.VMEM),
            pl.BlockSpec(memory_space=pltpu.VMEM),
            pl.BlockSpec(memory_space=pltpu.SMEM),
        ],
        out_specs=[
            pl.BlockSpec(memory_space=pltpu.VMEM),
            pl.BlockSpec(memory_space=pltpu.VMEM),
        ],
        out_shape=[
            jax.ShapeDtypeStruct((NCL, OUT), jnp.float32),
            jax.ShapeDtypeStruct((NCL, 8), jnp.float32),
        ],
        scratch_shapes=[
            pltpu.VMEM((N, OUT), jnp.float32),
            pltpu.VMEM((NCL, 1, OUT), jnp.float32),
            pltpu.VMEM((NCL, 8), jnp.float32),
        ],
    )(partials, rdeg, x, W_root, bias[None, :], gamma[None, :], beta[None, :],
      pos8, cid)


def kernel(x, edge_index, edge_attr, pos, batch, W, W_root, bias, gamma, beta):
    src = edge_index[0]
    dst = edge_index[1]

    Wf = jnp.transpose(W, (1, 0, 2)).reshape(IN, KK * OUT)
    x_trans = _xtrans(x, Wf)
    xt_flat = x_trans.reshape(N * KK, OUT)
    edata = _edgeprep(edge_attr.T, src[None, :], dst[None, :])

    c = jnp.clip(jnp.floor(pos / VOX).astype(jnp.int32), 0, G - 1)
    cid = c[:, 0] + c[:, 1] * G + c[:, 2] * (G * G)
    ei_flat = jnp.pad(edge_index.reshape(2 * E), (0, EPAD - 2 * E))

    zeros2 = jnp.zeros((NPAD, OUT), jnp.float32)
    zeros1 = jnp.zeros((NPAD,), jnp.float32)
    partials, degp = _edge_sc(edata.reshape(NBLK * EBW), xt_flat,
                              zeros2, zeros1)
    new_ei = _remap(cid, ei_flat)[:2 * E].reshape(2, E)
    degp = degp.reshape(NC, NPAD)
    rdeg = (1.0 / jnp.maximum(degp[0, :N] + degp[1, :N], 1.0))[:, None]

    pos8 = jnp.concatenate(
        [pos, jnp.ones((N, 1), jnp.float32), jnp.zeros((N, 4), jnp.float32)],
        axis=1)

    xp, pp8 = _post(partials, rdeg, x, W_root, bias, gamma, beta, pos8, cid)
    pp = pp8[:, :DIM]
    return xp, pp, new_ei
